# baseline (speedup 1.0000x reference)
import jax
import jax.numpy as jnp
from jax import lax
from jax.experimental import pallas as pl
from jax.experimental.pallas import tpu as pltpu
from jax.experimental.pallas import tpu_sc as plsc

V = 20000
E = 60000
D = 128
EPS = 0.01

NC = 2
NS = 16
NW = NC * NS
NT = 16
L = 16
EP = 61440
PERW = EP // NW
GCH = PERW // 128
PER = EP // NT
NV = PER // L
BITS = 8
BINS = 1 << BITS
PASSES = 4
CHUNKS = PER // 128
BLK = 4096


def _table_body(vsx_ref, vsy_ref, tb_ref):
    vx = vsx_ref[...]
    vy = vsy_ref[...]
    b = (vx < EPS) | (vx > 1.0 - EPS) | (vy < EPS) | (vy > 1.0 - EPS)
    tb_ref[...] = jnp.where(b, jnp.inf, 0.0)


def _build_table(vsx, vsy):
    return pl.pallas_call(
        _table_body,
        out_shape=jax.ShapeDtypeStruct((V,), jnp.float32),
    )(vsx, vsy)


def _sc_gather(image, srcp, dstp):
    nedges = srcp.shape[0]
    perw = nedges // NW
    csize = 128 if perw % 128 == 0 else 120
    nch_half = perw // csize
    nch = 2 * nch_half
    DEPTH = 4

    def body(image_hbm, srcp_hbm, dstp_hbm, gs_hbm, gd_hbm,
             idx_all, rows0, rows1, rows2, rows3,
             g0, g1, g2, g3, w0, w1, w2, w3):
        wid = lax.axis_index("s") * NC + lax.axis_index("c")
        base = wid * perw
        pltpu.sync_copy(srcp_hbm.at[pl.ds(base, perw)],
                        idx_all.at[pl.ds(0, perw)])
        pltpu.sync_copy(dstp_hbm.at[pl.ds(base, perw)],
                        idx_all.at[pl.ds(perw, perw)])
        rows = [rows0, rows1, rows2, rows3]
        gsem = [g0, g1, g2, g3]
        wsem = [w0, w1, w2, w3]

        def out_ref(c):
            if c < nch_half:
                return gs_hbm.at[pl.ds(base + c * csize, csize)]
            return gd_hbm.at[pl.ds(base + (c - nch_half) * csize, csize)]

        def issue_gather(c):
            return pltpu.async_copy(
                image_hbm.at[idx_all.at[pl.ds(c * csize, csize)]],
                rows[c % DEPTH], gsem[c % DEPTH])

        gops = [None] * nch
        wops = [None] * nch
        for c in range(min(2, nch)):
            gops[c] = issue_gather(c)
        for c in range(nch):
            gops[c].wait()
            wops[c] = pltpu.async_copy(rows[c % DEPTH], out_ref(c),
                                       wsem[c % DEPTH])
            nxt = c + 2
            if nxt < nch:
                if nxt - DEPTH >= 0:
                    wops[nxt - DEPTH].wait()
                gops[nxt] = issue_gather(nxt)
        for c in range(max(0, nch - DEPTH), nch):
            wops[c].wait()

    mesh = plsc.VectorSubcoreMesh(core_axis_name="c", subcore_axis_name="s")
    fn = pl.kernel(
        body,
        mesh=mesh,
        out_type=(jax.ShapeDtypeStruct((nedges, D), jnp.float32),
                  jax.ShapeDtypeStruct((nedges, D), jnp.float32)),
        scratch_types=[
            pltpu.VMEM((2 * perw,), jnp.int32),
            pltpu.VMEM((csize, D), jnp.float32),
            pltpu.VMEM((csize, D), jnp.float32),
            pltpu.VMEM((csize, D), jnp.float32),
            pltpu.VMEM((csize, D), jnp.float32),
            pltpu.SemaphoreType.DMA,
            pltpu.SemaphoreType.DMA,
            pltpu.SemaphoreType.DMA,
            pltpu.SemaphoreType.DMA,
            pltpu.SemaphoreType.DMA,
            pltpu.SemaphoreType.DMA,
            pltpu.SemaphoreType.DMA,
            pltpu.SemaphoreType.DMA,
        ],
    )
    return fn(image, srcp, dstp)


def _reduce_body(gs_ref, gd_ref, pr_ref):
    s = gs_ref[...]
    d = gd_ref[...]
    zs = s * s
    zd = d * d
    z = zs + zd
    zt = z.T
    acc = zt[0:8, :]
    for a in range(1, 16):
        acc = acc + zt[8 * a:8 * a + 8, :]
    u = acc[0:4, :] + acc[4:8, :]
    w = u[0:2, :] + u[2:4, :]
    p = w[0:1, :] + w[1:2, :]
    pr_ref[...] = p.reshape(pr_ref.shape[0])


def _reduce(gs, gd):
    nedges = gs.shape[0]
    blk = BLK if nedges % BLK == 0 else 3072
    return pl.pallas_call(
        _reduce_body,
        grid=(nedges // blk,),
        in_specs=[pl.BlockSpec((blk, D), lambda i: (i, 0)),
                  pl.BlockSpec((blk, D), lambda i: (i, 0))],
        out_specs=pl.BlockSpec((blk,), lambda i: (i,)),
        out_shape=jax.ShapeDtypeStruct((nedges,), jnp.float32),
    )(gs, gd)


def _iota16():
    return lax.iota(jnp.int32, L)


def _vreg_rank(keys_v, shift, i, sc16a):
    lanes = _iota16()
    k = keys_v[pl.ds(i * L, L)]
    d = lax.bitwise_and(lax.shift_right_logical(k, jnp.full((L,), shift, jnp.int32)),
                        jnp.full((L,), BINS - 1, jnp.int32))
    skey = lax.bitwise_or(lax.shift_left(d, jnp.full((L,), 4, jnp.int32)), lanes)
    sk, _ = plsc.sort_key_val(skey, lanes)
    ds = lax.shift_right_logical(sk, jnp.full((L,), 4, jnp.int32))
    ls = lax.bitwise_and(sk, jnp.full((L,), 15, jnp.int32))
    sc16a[...] = ds
    ds_prev = plsc.load_gather(sc16a, [jnp.maximum(lanes - 1, 0)])
    ds_next = plsc.load_gather(sc16a, [jnp.minimum(lanes + 1, L - 1)])
    chg = jnp.where(ds != ds_prev, lanes, 0)
    first = plsc.cummax(chg)
    r = lanes - first
    last = (ds != ds_next) | (lanes == L - 1)
    return ds, ls, r, last


def _sc_body(tb_hbm, pr_hbm, src_hbm, dst_hbm, prio_hbm, order_hbm,
             tb_v, s_v, d_v, pr_v, prio_v, keys_v, idx_v, pack_v, dest2d,
             hist_v, run_v, histall_v, sc16a, sc16b,
             a_k, a_i, b_k, b_i, ghist):
    wid = lax.axis_index("s")
    base = wid * PER
    lanes = _iota16()

    pltpu.sync_copy(tb_hbm, tb_v)
    pltpu.sync_copy(pr_hbm.at[pl.ds(base, PER)], pr_v)
    pltpu.sync_copy(src_hbm.at[pl.ds(base, PER)], s_v)
    pltpu.sync_copy(dst_hbm.at[pl.ds(base, PER)], d_v)

    def gather_body(i, _):
        s = s_v[pl.ds(i * L, L)]
        d = d_v[pl.ds(i * L, L)]
        m = plsc.load_gather(tb_v, [s]) + plsc.load_gather(tb_v, [d])
        p = m + pr_v[pl.ds(i * L, L)]
        prio_v[pl.ds(i * L, L)] = p
        gidx = base + i * L + lanes
        kbits = plsc.bitcast(p, jnp.int32)
        keys_v[pl.ds(i * L, L)] = jnp.where(gidx < E, kbits,
                                            jnp.full((L,), -1, jnp.int32))
        idx_v[pl.ds(i * L, L)] = gidx
        return 0

    lax.fori_loop(0, NV, gather_body, 0)
    pltpu.sync_copy(prio_v, prio_hbm.at[pl.ds(base, PER)])

    bufs = [(a_k, a_i, b_k, b_i), (b_k, b_i, a_k, a_i)]
    for p in range(PASSES):
        src_k, src_i, dst_k, dst_i = bufs[p % 2]
        shift = p * BITS
        if p > 0:
            pltpu.sync_copy(src_k.at[pl.ds(base, PER)], keys_v)
            pltpu.sync_copy(src_i.at[pl.ds(base, PER)], idx_v)

        for c in range(BINS // L):
            hist_v[pl.ds(c * L, L)] = jnp.zeros((L,), jnp.int32)

        def hist_body(i, _):
            ds, ls, r, last = _vreg_rank(keys_v, shift, i, sc16a)
            c8 = jnp.full((L,), 8, jnp.int32)
            c4 = jnp.full((L,), 4, jnp.int32)
            c16 = jnp.full((L,), 16, jnp.int32)
            pack = lax.bitwise_or(
                lax.bitwise_or(lax.shift_left(ds, c8), lax.shift_left(ls, c4)),
                r)
            pack = lax.bitwise_or(
                pack, lax.shift_left(jnp.where(last, 1, 0), c16))
            pack_v[pl.ds(i * L, L)] = pack
            plsc.addupdate_scatter(hist_v, [ds], r + 1, mask=last)
            return 0

        lax.fori_loop(0, NV, hist_body, 0)
        pltpu.sync_copy(hist_v, ghist.at[wid])
        plsc.subcore_barrier()
        pltpu.sync_copy(ghist, histall_v)

        def base_chunk(c, carry):
            def row_body(t, acc):
                tot, below = acc
                h = histall_v[t, pl.ds(c * L, L)]
                m = jnp.where(t < wid, 1, 0)
                return (tot + h, below + h * m)

            tot, below = lax.fori_loop(0, NT, row_body,
                                       (jnp.zeros((L,), jnp.int32),
                                        jnp.zeros((L,), jnp.int32)))
            inc = plsc.cumsum(tot)
            excl = inc - tot
            run_v[pl.ds(c * L, L)] = excl + below + carry
            return carry + jnp.sum(tot)

        lax.fori_loop(0, BINS // L, base_chunk, jnp.int32(0))

        def rank_body(i, _):
            c4 = jnp.full((L,), 4, jnp.int32)
            c8 = jnp.full((L,), 8, jnp.int32)
            c16 = jnp.full((L,), 16, jnp.int32)
            m15 = jnp.full((L,), 15, jnp.int32)
            pack = pack_v[pl.ds(i * L, L)]
            r = lax.bitwise_and(pack, m15)
            ls = lax.bitwise_and(lax.shift_right_logical(pack, c4), m15)
            ds = lax.bitwise_and(lax.shift_right_logical(pack, c8),
                                 jnp.full((L,), BINS - 1, jnp.int32))
            last = lax.bitwise_and(lax.shift_right_logical(pack, c16),
                                   jnp.full((L,), 1, jnp.int32)) != 0
            baseg = plsc.load_gather(run_v, [ds])
            dsorted = baseg + r
            plsc.store_scatter(sc16b, [ls], dsorted)
            dlane = sc16b[...]
            dest2d[i // 8, pl.ds((i % 8) * L, L)] = dlane
            plsc.addupdate_scatter(run_v, [ds], r + 1, mask=last)
            return 0

        lax.fori_loop(0, NV, rank_body, 0)

        for c in range(CHUNKS):
            pltpu.sync_copy(keys_v.at[pl.ds(c * 128, 128)],
                            dst_k.at[dest2d.at[c]])
            pltpu.sync_copy(idx_v.at[pl.ds(c * 128, 128)],
                            dst_i.at[dest2d.at[c]])
        plsc.subcore_barrier()

    pltpu.sync_copy(a_i.at[pl.ds(base, PER)], idx_v)
    pltpu.sync_copy(idx_v, order_hbm.at[pl.ds(base, PER)])


def _sc_sort(tb, pr, src, dst):
    mesh = plsc.VectorSubcoreMesh(core_axis_name="c", subcore_axis_name="s",
                                  num_cores=1)
    fn = pl.kernel(
        _sc_body,
        mesh=mesh,
        compiler_params=pltpu.CompilerParams(needs_layout_passes=False),
        out_type=(jax.ShapeDtypeStruct((EP,), jnp.float32),
                  jax.ShapeDtypeStruct((EP,), jnp.int32)),
        scratch_types=[
            pltpu.VMEM((V,), jnp.float32),
            pltpu.VMEM((PER,), jnp.int32),
            pltpu.VMEM((PER,), jnp.int32),
            pltpu.VMEM((PER,), jnp.float32),
            pltpu.VMEM((PER,), jnp.float32),
            pltpu.VMEM((PER,), jnp.int32),
            pltpu.VMEM((PER,), jnp.int32),
            pltpu.VMEM((PER,), jnp.int32),
            pltpu.VMEM((CHUNKS, 128), jnp.int32),
            pltpu.VMEM((BINS,), jnp.int32),
            pltpu.VMEM((BINS,), jnp.int32),
            pltpu.VMEM((NT, BINS), jnp.int32),
            pltpu.VMEM((L,), jnp.int32),
            pltpu.VMEM((L,), jnp.int32),
            pltpu.VMEM_SHARED((EP,), jnp.int32),
            pltpu.VMEM_SHARED((EP,), jnp.int32),
            pltpu.VMEM_SHARED((EP,), jnp.int32),
            pltpu.VMEM_SHARED((EP,), jnp.int32),
            pltpu.VMEM_SHARED((NT, BINS), jnp.int32),
        ],
    )
    return fn(tb, pr, src, dst)


def kernel(image, edges, vs):
    tb = _build_table(vs[:, 0], vs[:, 1])
    pad = jnp.arange(EP - E, dtype=jnp.int32)
    src = jnp.concatenate([edges[0], pad])
    dst = jnp.concatenate([edges[1], pad])
    half = EP // 2
    gs1, gd1 = _sc_gather(image, src[:half], dst[:half])
    gs2, gd2 = _sc_gather(image, src[half:], dst[half:])
    pr1 = _reduce(gs1, gd1)
    pr2 = _reduce(gs2, gd2)
    pr = jnp.concatenate([pr1, pr2])
    prio, order = _sc_sort(tb, pr, src, dst)
    return prio[:E], order[:E]


# speedup vs baseline: 1.0202x; 1.0202x over previous
import jax
import jax.numpy as jnp
from jax import lax
from jax.experimental import pallas as pl
from jax.experimental.pallas import tpu as pltpu
from jax.experimental.pallas import tpu_sc as plsc

V = 20000
E = 60000
D = 128
EPS = 0.01

NC = 2
NS = 16
NW = NC * NS
NT = 16
L = 16
EP = 61440
PERW = EP // NW
GCH = PERW // 128
PER = EP // NT
NV = PER // L
BITS = 8
BINS = 1 << BITS
PASSES = 4
CHUNKS = PER // 128
BLK = 4096


def _table_body(vsx_ref, vsy_ref, tb_ref):
    vx = vsx_ref[...]
    vy = vsy_ref[...]
    b = (vx < EPS) | (vx > 1.0 - EPS) | (vy < EPS) | (vy > 1.0 - EPS)
    tb_ref[...] = jnp.where(b, jnp.inf, 0.0)


def _build_table(vsx, vsy):
    return pl.pallas_call(
        _table_body,
        out_shape=jax.ShapeDtypeStruct((V,), jnp.float32),
    )(vsx, vsy)


def _sc_gather(image, srcp, dstp):
    nedges = srcp.shape[0]
    perw = nedges // NW
    csize = 128 if perw % 128 == 0 else 120
    nch_half = perw // csize
    nch = 2 * nch_half
    DEPTH = 4

    def body(image_hbm, srcp_hbm, dstp_hbm, gs_hbm, gd_hbm,
             idx_all, rows0, rows1, rows2, rows3,
             g0, g1, g2, g3, w0, w1, w2, w3):
        wid = lax.axis_index("s") * NC + lax.axis_index("c")
        base = wid * perw
        pltpu.sync_copy(srcp_hbm.at[pl.ds(base, perw)],
                        idx_all.at[pl.ds(0, perw)])
        pltpu.sync_copy(dstp_hbm.at[pl.ds(base, perw)],
                        idx_all.at[pl.ds(perw, perw)])
        rows = [rows0, rows1, rows2, rows3]
        gsem = [g0, g1, g2, g3]
        wsem = [w0, w1, w2, w3]

        def out_ref(c):
            if c < nch_half:
                return gs_hbm.at[pl.ds(base + c * csize, csize)]
            return gd_hbm.at[pl.ds(base + (c - nch_half) * csize, csize)]

        def issue_gather(c):
            return pltpu.async_copy(
                image_hbm.at[idx_all.at[pl.ds(c * csize, csize)]],
                rows[c % DEPTH], gsem[c % DEPTH])

        gops = [None] * nch
        wops = [None] * nch
        for c in range(min(2, nch)):
            gops[c] = issue_gather(c)
        for c in range(nch):
            gops[c].wait()
            wops[c] = pltpu.async_copy(rows[c % DEPTH], out_ref(c),
                                       wsem[c % DEPTH])
            nxt = c + 2
            if nxt < nch:
                if nxt - DEPTH >= 0:
                    wops[nxt - DEPTH].wait()
                gops[nxt] = issue_gather(nxt)
        for c in range(max(0, nch - DEPTH), nch):
            wops[c].wait()

    mesh = plsc.VectorSubcoreMesh(core_axis_name="c", subcore_axis_name="s")
    fn = pl.kernel(
        body,
        mesh=mesh,
        out_type=(jax.ShapeDtypeStruct((nedges, D), jnp.float32),
                  jax.ShapeDtypeStruct((nedges, D), jnp.float32)),
        scratch_types=[
            pltpu.VMEM((2 * perw,), jnp.int32),
            pltpu.VMEM((csize, D), jnp.float32),
            pltpu.VMEM((csize, D), jnp.float32),
            pltpu.VMEM((csize, D), jnp.float32),
            pltpu.VMEM((csize, D), jnp.float32),
            pltpu.SemaphoreType.DMA,
            pltpu.SemaphoreType.DMA,
            pltpu.SemaphoreType.DMA,
            pltpu.SemaphoreType.DMA,
            pltpu.SemaphoreType.DMA,
            pltpu.SemaphoreType.DMA,
            pltpu.SemaphoreType.DMA,
            pltpu.SemaphoreType.DMA,
        ],
    )
    return fn(image, srcp, dstp)


def _reduce_body(gs_ref, gd_ref, pr_ref):
    s = gs_ref[...]
    d = gd_ref[...]
    zs = s * s
    zd = d * d
    z = zs + zd
    zt = z.T
    acc = zt[0:8, :]
    for a in range(1, 16):
        acc = acc + zt[8 * a:8 * a + 8, :]
    u = acc[0:4, :] + acc[4:8, :]
    w = u[0:2, :] + u[2:4, :]
    p = w[0:1, :] + w[1:2, :]
    pr_ref[...] = p.reshape(pr_ref.shape[0])


def _reduce(gs, gd):
    nedges = gs.shape[0]
    blk = BLK if nedges % BLK == 0 else 3072
    return pl.pallas_call(
        _reduce_body,
        grid=(nedges // blk,),
        in_specs=[pl.BlockSpec((blk, D), lambda i: (i, 0)),
                  pl.BlockSpec((blk, D), lambda i: (i, 0))],
        out_specs=pl.BlockSpec((blk,), lambda i: (i,)),
        out_shape=jax.ShapeDtypeStruct((nedges,), jnp.float32),
    )(gs, gd)


def _iota16():
    return lax.iota(jnp.int32, L)


def _vreg_rank(keys_v, shift, i, sc16a):
    lanes = _iota16()
    k = keys_v[pl.ds(i * L, L)]
    d = lax.bitwise_and(lax.shift_right_logical(k, jnp.full((L,), shift, jnp.int32)),
                        jnp.full((L,), BINS - 1, jnp.int32))
    skey = lax.bitwise_or(lax.shift_left(d, jnp.full((L,), 4, jnp.int32)), lanes)
    sk, _ = plsc.sort_key_val(skey, lanes)
    ds = lax.shift_right_logical(sk, jnp.full((L,), 4, jnp.int32))
    ls = lax.bitwise_and(sk, jnp.full((L,), 15, jnp.int32))
    sc16a[...] = ds
    ds_prev = plsc.load_gather(sc16a, [jnp.maximum(lanes - 1, 0)])
    ds_next = plsc.load_gather(sc16a, [jnp.minimum(lanes + 1, L - 1)])
    chg = jnp.where(ds != ds_prev, lanes, 0)
    first = plsc.cummax(chg)
    r = lanes - first
    last = (ds != ds_next) | (lanes == L - 1)
    return ds, ls, r, last


def _sc_body(tb_hbm, pr_hbm, src_hbm, dst_hbm, prio_hbm, order_hbm,
             tb_v, s_v, d_v, pr_v, prio_v, keys_v, idx_v, pack_v, dest2d,
             hist_v, run_v, histall_v, sc16a, sc16b,
             a_k, a_i, b_k, b_i, ghist):
    wid = lax.axis_index("s")
    base = wid * PER
    lanes = _iota16()

    pltpu.sync_copy(tb_hbm, tb_v)
    pltpu.sync_copy(pr_hbm.at[pl.ds(base, PER)], pr_v)
    pltpu.sync_copy(src_hbm.at[pl.ds(base, PER)], s_v)
    pltpu.sync_copy(dst_hbm.at[pl.ds(base, PER)], d_v)

    def build_keys(i):
        s = s_v[pl.ds(i * L, L)]
        d = d_v[pl.ds(i * L, L)]
        m = plsc.load_gather(tb_v, [s]) + plsc.load_gather(tb_v, [d])
        p = m + pr_v[pl.ds(i * L, L)]
        prio_v[pl.ds(i * L, L)] = p
        gidx = base + i * L + lanes
        kbits = plsc.bitcast(p, jnp.int32)
        keys_v[pl.ds(i * L, L)] = jnp.where(gidx < E, kbits,
                                            jnp.full((L,), -1, jnp.int32))
        idx_v[pl.ds(i * L, L)] = gidx

    bufs = [(a_k, a_i, b_k, b_i), (b_k, b_i, a_k, a_i)]
    for p in range(PASSES):
        src_k, src_i, dst_k, dst_i = bufs[p % 2]
        shift = p * BITS
        if p > 0:
            pltpu.sync_copy(src_k.at[pl.ds(base, PER)], keys_v)
            pltpu.sync_copy(src_i.at[pl.ds(base, PER)], idx_v)

        for c in range(BINS // L):
            hist_v[pl.ds(c * L, L)] = jnp.zeros((L,), jnp.int32)

        def hist_one(i):
            if p == 0:
                build_keys(i)
            ds, ls, r, last = _vreg_rank(keys_v, shift, i, sc16a)
            c8 = jnp.full((L,), 8, jnp.int32)
            c4 = jnp.full((L,), 4, jnp.int32)
            c16 = jnp.full((L,), 16, jnp.int32)
            pack = lax.bitwise_or(
                lax.bitwise_or(lax.shift_left(ds, c8), lax.shift_left(ls, c4)),
                r)
            pack = lax.bitwise_or(
                pack, lax.shift_left(jnp.where(last, 1, 0), c16))
            pack_v[pl.ds(i * L, L)] = pack
            plsc.addupdate_scatter(hist_v, [ds], r + 1, mask=last)

        def hist_body(i, _):
            hist_one(2 * i)
            hist_one(2 * i + 1)
            return 0

        lax.fori_loop(0, NV // 2, hist_body, 0)
        if p == 0:
            pltpu.sync_copy(prio_v, prio_hbm.at[pl.ds(base, PER)])
        pltpu.sync_copy(hist_v, ghist.at[wid])
        plsc.subcore_barrier()
        pltpu.sync_copy(ghist, histall_v)

        def base_chunk(c, carry):
            def row_body(t, acc):
                tot, below = acc
                h = histall_v[t, pl.ds(c * L, L)]
                m = jnp.where(t < wid, 1, 0)
                return (tot + h, below + h * m)

            tot, below = lax.fori_loop(0, NT, row_body,
                                       (jnp.zeros((L,), jnp.int32),
                                        jnp.zeros((L,), jnp.int32)))
            inc = plsc.cumsum(tot)
            excl = inc - tot
            run_v[pl.ds(c * L, L)] = excl + below + carry
            return carry + jnp.sum(tot)

        lax.fori_loop(0, BINS // L, base_chunk, jnp.int32(0))

        def rank_one(i):
            c4 = jnp.full((L,), 4, jnp.int32)
            c8 = jnp.full((L,), 8, jnp.int32)
            c16 = jnp.full((L,), 16, jnp.int32)
            m15 = jnp.full((L,), 15, jnp.int32)
            pack = pack_v[pl.ds(i * L, L)]
            r = lax.bitwise_and(pack, m15)
            ls = lax.bitwise_and(lax.shift_right_logical(pack, c4), m15)
            ds = lax.bitwise_and(lax.shift_right_logical(pack, c8),
                                 jnp.full((L,), BINS - 1, jnp.int32))
            last = lax.bitwise_and(lax.shift_right_logical(pack, c16),
                                   jnp.full((L,), 1, jnp.int32)) != 0
            baseg = plsc.load_gather(run_v, [ds])
            dsorted = baseg + r
            plsc.store_scatter(sc16b, [ls], dsorted)
            dlane = sc16b[...]
            dest2d[i // 8, pl.ds((i % 8) * L, L)] = dlane
            plsc.addupdate_scatter(run_v, [ds], r + 1, mask=last)

        def rank_body(i, _):
            rank_one(2 * i)
            rank_one(2 * i + 1)
            return 0

        lax.fori_loop(0, NV // 2, rank_body, 0)

        for c in range(CHUNKS):
            pltpu.sync_copy(keys_v.at[pl.ds(c * 128, 128)],
                            dst_k.at[dest2d.at[c]])
            pltpu.sync_copy(idx_v.at[pl.ds(c * 128, 128)],
                            dst_i.at[dest2d.at[c]])
        plsc.subcore_barrier()

    pltpu.sync_copy(a_i.at[pl.ds(base, PER)], idx_v)
    pltpu.sync_copy(idx_v, order_hbm.at[pl.ds(base, PER)])


def _sc_sort(tb, pr, src, dst):
    mesh = plsc.VectorSubcoreMesh(core_axis_name="c", subcore_axis_name="s",
                                  num_cores=1)
    fn = pl.kernel(
        _sc_body,
        mesh=mesh,
        compiler_params=pltpu.CompilerParams(needs_layout_passes=False),
        out_type=(jax.ShapeDtypeStruct((EP,), jnp.float32),
                  jax.ShapeDtypeStruct((EP,), jnp.int32)),
        scratch_types=[
            pltpu.VMEM((V,), jnp.float32),
            pltpu.VMEM((PER,), jnp.int32),
            pltpu.VMEM((PER,), jnp.int32),
            pltpu.VMEM((PER,), jnp.float32),
            pltpu.VMEM((PER,), jnp.float32),
            pltpu.VMEM((PER,), jnp.int32),
            pltpu.VMEM((PER,), jnp.int32),
            pltpu.VMEM((PER,), jnp.int32),
            pltpu.VMEM((CHUNKS, 128), jnp.int32),
            pltpu.VMEM((BINS,), jnp.int32),
            pltpu.VMEM((BINS,), jnp.int32),
            pltpu.VMEM((NT, BINS), jnp.int32),
            pltpu.VMEM((L,), jnp.int32),
            pltpu.VMEM((L,), jnp.int32),
            pltpu.VMEM_SHARED((EP,), jnp.int32),
            pltpu.VMEM_SHARED((EP,), jnp.int32),
            pltpu.VMEM_SHARED((EP,), jnp.int32),
            pltpu.VMEM_SHARED((EP,), jnp.int32),
            pltpu.VMEM_SHARED((NT, BINS), jnp.int32),
        ],
    )
    return fn(tb, pr, src, dst)


def kernel(image, edges, vs):
    tb = _build_table(vs[:, 0], vs[:, 1])
    pad = jnp.arange(EP - E, dtype=jnp.int32)
    src = jnp.concatenate([edges[0], pad])
    dst = jnp.concatenate([edges[1], pad])
    gs, gd = _sc_gather(image, src, dst)
    pr = _reduce(gs, gd)
    prio, order = _sc_sort(tb, pr, src, dst)
    return prio[:E], order[:E]


# speedup vs baseline: 1.1220x; 1.0999x over previous
import jax
import jax.numpy as jnp
from jax import lax
from jax.experimental import pallas as pl
from jax.experimental.pallas import tpu as pltpu
from jax.experimental.pallas import tpu_sc as plsc

V = 20000
E = 60000
D = 128
EPS = 0.01

NC = 2
NS = 16
NW = NC * NS
NT = 16
L = 16
EP = 61440
PERW = EP // NW
GCH = PERW // 128
PER = EP // NT
NV = PER // L
BITS = 8
BINS = 1 << BITS
PASSES = 4
CHUNKS = PER // 128
BLK = 4096


def _table_body(vsx_ref, vsy_ref, tb_ref):
    vx = vsx_ref[...]
    vy = vsy_ref[...]
    b = (vx < EPS) | (vx > 1.0 - EPS) | (vy < EPS) | (vy > 1.0 - EPS)
    tb_ref[...] = jnp.where(b, jnp.inf, 0.0)


def _build_table(vsx, vsy):
    return pl.pallas_call(
        _table_body,
        out_shape=jax.ShapeDtypeStruct((V,), jnp.float32),
    )(vsx, vsy)


def _sc_gather(image, srcp, dstp):
    nedges = srcp.shape[0]
    perw = nedges // NW
    csize = 128 if perw % 128 == 0 else 120
    nch_half = perw // csize
    nch = 2 * nch_half
    DEPTH = 4

    def body(image_hbm, srcp_hbm, dstp_hbm, gs_hbm, gd_hbm,
             idx_all, rows0, rows1, rows2, rows3,
             g0, g1, g2, g3, w0, w1, w2, w3):
        wid = lax.axis_index("s") * NC + lax.axis_index("c")
        base = wid * perw
        pltpu.sync_copy(srcp_hbm.at[pl.ds(base, perw)],
                        idx_all.at[pl.ds(0, perw)])
        pltpu.sync_copy(dstp_hbm.at[pl.ds(base, perw)],
                        idx_all.at[pl.ds(perw, perw)])
        rows = [rows0, rows1, rows2, rows3]
        gsem = [g0, g1, g2, g3]
        wsem = [w0, w1, w2, w3]

        def out_ref(c):
            if c < nch_half:
                return gs_hbm.at[pl.ds(base + c * csize, csize)]
            return gd_hbm.at[pl.ds(base + (c - nch_half) * csize, csize)]

        def issue_gather(c):
            return pltpu.async_copy(
                image_hbm.at[idx_all.at[pl.ds(c * csize, csize)]],
                rows[c % DEPTH], gsem[c % DEPTH])

        gops = [None] * nch
        wops = [None] * nch
        for c in range(min(2, nch)):
            gops[c] = issue_gather(c)
        for c in range(nch):
            gops[c].wait()
            wops[c] = pltpu.async_copy(rows[c % DEPTH], out_ref(c),
                                       wsem[c % DEPTH])
            nxt = c + 2
            if nxt < nch:
                if nxt - DEPTH >= 0:
                    wops[nxt - DEPTH].wait()
                gops[nxt] = issue_gather(nxt)
        for c in range(max(0, nch - DEPTH), nch):
            wops[c].wait()

    mesh = plsc.VectorSubcoreMesh(core_axis_name="c", subcore_axis_name="s")
    fn = pl.kernel(
        body,
        mesh=mesh,
        out_type=(jax.ShapeDtypeStruct((nedges, D), jnp.float32),
                  jax.ShapeDtypeStruct((nedges, D), jnp.float32)),
        scratch_types=[
            pltpu.VMEM((2 * perw,), jnp.int32),
            pltpu.VMEM((csize, D), jnp.float32),
            pltpu.VMEM((csize, D), jnp.float32),
            pltpu.VMEM((csize, D), jnp.float32),
            pltpu.VMEM((csize, D), jnp.float32),
            pltpu.SemaphoreType.DMA,
            pltpu.SemaphoreType.DMA,
            pltpu.SemaphoreType.DMA,
            pltpu.SemaphoreType.DMA,
            pltpu.SemaphoreType.DMA,
            pltpu.SemaphoreType.DMA,
            pltpu.SemaphoreType.DMA,
            pltpu.SemaphoreType.DMA,
        ],
    )
    return fn(image, srcp, dstp)


def _reduce_body(gs_ref, gd_ref, pr_ref):
    s = gs_ref[...]
    d = gd_ref[...]
    zs = s * s
    zd = d * d
    z = zs + zd
    zt = z.T
    acc = zt[0:8, :]
    for a in range(1, 16):
        acc = acc + zt[8 * a:8 * a + 8, :]
    u = acc[0:4, :] + acc[4:8, :]
    w = u[0:2, :] + u[2:4, :]
    p = w[0:1, :] + w[1:2, :]
    pr_ref[...] = p.reshape(pr_ref.shape[0])


def _reduce(gs, gd):
    nedges = gs.shape[0]
    blk = BLK if nedges % BLK == 0 else 3072
    return pl.pallas_call(
        _reduce_body,
        grid=(nedges // blk,),
        in_specs=[pl.BlockSpec((blk, D), lambda i: (i, 0)),
                  pl.BlockSpec((blk, D), lambda i: (i, 0))],
        out_specs=pl.BlockSpec((blk,), lambda i: (i,)),
        out_shape=jax.ShapeDtypeStruct((nedges,), jnp.float32),
    )(gs, gd)


def _iota16():
    return lax.iota(jnp.int32, L)


def _vreg_rank(keys_v, shift, i, sc16a):
    lanes = _iota16()
    k = keys_v[pl.ds(i * L, L)]
    d = lax.bitwise_and(lax.shift_right_logical(k, jnp.full((L,), shift, jnp.int32)),
                        jnp.full((L,), BINS - 1, jnp.int32))
    skey = lax.bitwise_or(lax.shift_left(d, jnp.full((L,), 4, jnp.int32)), lanes)
    sk, _ = plsc.sort_key_val(skey, lanes)
    ds = lax.shift_right_logical(sk, jnp.full((L,), 4, jnp.int32))
    ls = lax.bitwise_and(sk, jnp.full((L,), 15, jnp.int32))
    sc16a[...] = ds
    ds_prev = plsc.load_gather(sc16a, [jnp.maximum(lanes - 1, 0)])
    ds_next = plsc.load_gather(sc16a, [jnp.minimum(lanes + 1, L - 1)])
    chg = jnp.where(ds != ds_prev, lanes, 0)
    first = plsc.cummax(chg)
    r = lanes - first
    last = (ds != ds_next) | (lanes == L - 1)
    return ds, ls, r, last


def _sc_body(tb_hbm, pr_hbm, src_hbm, dst_hbm, prio_hbm, order_hbm,
             tb_v, s_v, d_v, pr_v, prio_v, keys_v, idx_v, pack_v, dest2d,
             hist_v, run_v, histall_v, sc16a, sc16b,
             a_k, a_i, b_k, b_i, ghist):
    wid = lax.axis_index("s")
    base = wid * PER
    lanes = _iota16()

    pltpu.sync_copy(tb_hbm, tb_v)
    pltpu.sync_copy(pr_hbm.at[pl.ds(base, PER)], pr_v)
    pltpu.sync_copy(src_hbm.at[pl.ds(base, PER)], s_v)
    pltpu.sync_copy(dst_hbm.at[pl.ds(base, PER)], d_v)

    def build_keys(i):
        s = s_v[pl.ds(i * L, L)]
        d = d_v[pl.ds(i * L, L)]
        m = plsc.load_gather(tb_v, [s]) + plsc.load_gather(tb_v, [d])
        p = m + pr_v[pl.ds(i * L, L)]
        prio_v[pl.ds(i * L, L)] = p
        gidx = base + i * L + lanes
        kbits = plsc.bitcast(p, jnp.int32)
        keys_v[pl.ds(i * L, L)] = jnp.where(gidx < E, kbits,
                                            jnp.full((L,), -1, jnp.int32))
        idx_v[pl.ds(i * L, L)] = gidx

    bufs = [(a_k, a_i, b_k, b_i), (b_k, b_i, a_k, a_i)]
    for p in range(PASSES):
        src_k, src_i, dst_k, dst_i = bufs[p % 2]
        shift = p * BITS
        if p > 0:
            pltpu.sync_copy(src_k.at[pl.ds(base, PER)], keys_v)
            pltpu.sync_copy(src_i.at[pl.ds(base, PER)], idx_v)

        for c in range(BINS // L):
            hist_v[pl.ds(c * L, L)] = jnp.zeros((L,), jnp.int32)

        def hist_one(i):
            if p == 0:
                build_keys(i)
            ds, ls, r, last = _vreg_rank(keys_v, shift, i, sc16a)
            c8 = jnp.full((L,), 8, jnp.int32)
            c4 = jnp.full((L,), 4, jnp.int32)
            c16 = jnp.full((L,), 16, jnp.int32)
            pack = lax.bitwise_or(
                lax.bitwise_or(lax.shift_left(ds, c8), lax.shift_left(ls, c4)),
                r)
            pack = lax.bitwise_or(
                pack, lax.shift_left(jnp.where(last, 1, 0), c16))
            pack_v[pl.ds(i * L, L)] = pack
            plsc.addupdate_scatter(hist_v, [ds], r + 1, mask=last)

        def hist_body(i, _):
            hist_one(2 * i)
            hist_one(2 * i + 1)
            return 0

        lax.fori_loop(0, NV // 2, hist_body, 0)
        if p == 0:
            pltpu.sync_copy(prio_v, prio_hbm.at[pl.ds(base, PER)])
        pltpu.sync_copy(hist_v, ghist.at[wid])
        plsc.subcore_barrier()
        pltpu.sync_copy(ghist, histall_v)

        def base_chunk(c, carry):
            def row_body(t, acc):
                tot, below = acc
                h = histall_v[t, pl.ds(c * L, L)]
                m = jnp.where(t < wid, 1, 0)
                return (tot + h, below + h * m)

            tot, below = lax.fori_loop(0, NT, row_body,
                                       (jnp.zeros((L,), jnp.int32),
                                        jnp.zeros((L,), jnp.int32)))
            inc = plsc.cumsum(tot)
            excl = inc - tot
            run_v[pl.ds(c * L, L)] = excl + below + carry
            return carry + jnp.sum(tot)

        lax.fori_loop(0, BINS // L, base_chunk, jnp.int32(0))

        def rank_one(i):
            c4 = jnp.full((L,), 4, jnp.int32)
            c8 = jnp.full((L,), 8, jnp.int32)
            c16 = jnp.full((L,), 16, jnp.int32)
            m15 = jnp.full((L,), 15, jnp.int32)
            pack = pack_v[pl.ds(i * L, L)]
            r = lax.bitwise_and(pack, m15)
            ls = lax.bitwise_and(lax.shift_right_logical(pack, c4), m15)
            ds = lax.bitwise_and(lax.shift_right_logical(pack, c8),
                                 jnp.full((L,), BINS - 1, jnp.int32))
            last = lax.bitwise_and(lax.shift_right_logical(pack, c16),
                                   jnp.full((L,), 1, jnp.int32)) != 0
            baseg = plsc.load_gather(run_v, [ds])
            dsorted = baseg + r
            plsc.store_scatter(sc16b, [ls], dsorted)
            dlane = sc16b[...]
            dest2d[pl.ds(i * L, L)] = dlane
            plsc.addupdate_scatter(run_v, [ds], r + 1, mask=last)

        def rank_body(i, _):
            rank_one(2 * i)
            rank_one(2 * i + 1)
            return 0

        lax.fori_loop(0, NV // 2, rank_body, 0)

        pltpu.sync_copy(keys_v, dst_k.at[dest2d])
        pltpu.sync_copy(idx_v, dst_i.at[dest2d])
        plsc.subcore_barrier()

    pltpu.sync_copy(a_i.at[pl.ds(base, PER)], idx_v)
    pltpu.sync_copy(idx_v, order_hbm.at[pl.ds(base, PER)])


def _sc_sort(tb, pr, src, dst):
    mesh = plsc.VectorSubcoreMesh(core_axis_name="c", subcore_axis_name="s",
                                  num_cores=1)
    fn = pl.kernel(
        _sc_body,
        mesh=mesh,
        compiler_params=pltpu.CompilerParams(needs_layout_passes=False),
        out_type=(jax.ShapeDtypeStruct((EP,), jnp.float32),
                  jax.ShapeDtypeStruct((EP,), jnp.int32)),
        scratch_types=[
            pltpu.VMEM((V,), jnp.float32),
            pltpu.VMEM((PER,), jnp.int32),
            pltpu.VMEM((PER,), jnp.int32),
            pltpu.VMEM((PER,), jnp.float32),
            pltpu.VMEM((PER,), jnp.float32),
            pltpu.VMEM((PER,), jnp.int32),
            pltpu.VMEM((PER,), jnp.int32),
            pltpu.VMEM((PER,), jnp.int32),
            pltpu.VMEM((PER,), jnp.int32),
            pltpu.VMEM((BINS,), jnp.int32),
            pltpu.VMEM((BINS,), jnp.int32),
            pltpu.VMEM((NT, BINS), jnp.int32),
            pltpu.VMEM((L,), jnp.int32),
            pltpu.VMEM((L,), jnp.int32),
            pltpu.VMEM_SHARED((EP,), jnp.int32),
            pltpu.VMEM_SHARED((EP,), jnp.int32),
            pltpu.VMEM_SHARED((EP,), jnp.int32),
            pltpu.VMEM_SHARED((EP,), jnp.int32),
            pltpu.VMEM_SHARED((NT, BINS), jnp.int32),
        ],
    )
    return fn(tb, pr, src, dst)


def kernel(image, edges, vs):
    tb = _build_table(vs[:, 0], vs[:, 1])
    pad = jnp.arange(EP - E, dtype=jnp.int32)
    src = jnp.concatenate([edges[0], pad])
    dst = jnp.concatenate([edges[1], pad])
    gs, gd = _sc_gather(image, src, dst)
    pr = _reduce(gs, gd)
    prio, order = _sc_sort(tb, pr, src, dst)
    return prio[:E], order[:E]
